# baseline (speedup 1.0000x reference)
import functools

import jax
import jax.numpy as jnp
from jax import lax
from jax.experimental import pallas as pl
from jax.experimental.pallas import tpu as pltpu
from jax.experimental.pallas import tpu_sc as plsc

N_NODES = 10000
DEG = 16
D = 256
B = 10000
BP = 10240
NC = 2
NS = 16
NW = NC * NS
BPW = BP // NW
CH = 16
NCHUNK = BPW // CH
NCH0 = 18
NCH1 = 2 * NCHUNK - NCH0


def _sc_gather_kernel(nodes_h, adj_h, emb_h, self_h, nsum_h,
                      idxc, adjc0, adjc1, nidx_all,
                      sbuf0, sbuf1, nbA0, nbA1, nbB0, nbB1, mbuf0, mbuf1,
                      sga0, sga1, sgA0, sgA1, sgB0, sgB1, sgs0, sgs1,
                      swm0, swm1, sws0, sws1):
    cid = lax.axis_index("c")
    sid = lax.axis_index("s")
    nch = jnp.where(cid == 0, NCH0, NCH1)
    base = sid * (2 * BPW) + cid * (NCH0 * CH)

    adjc = (adjc0, adjc1)
    sbuf = (sbuf0, sbuf1)
    nbA = (nbA0, nbA1)
    nbB = (nbB0, nbB1)
    mbuf = (mbuf0, mbuf1)
    sga = (sga0, sga1)
    sgA = (sgA0, sgA1)
    sgB = (sgB0, sgB1)
    sgs = (sgs0, sgs1)
    swm = (swm0, swm1)
    sws = (sws0, sws1)

    pltpu.sync_copy(nodes_h.at[pl.ds(base, NCH1 * CH)], idxc)

    def ids(c):
        return idxc.at[pl.ds(c * CH, CH)]

    def adj_dma(c, k):
        return pltpu.make_async_copy(adj_h.at[ids(c)], adjc[k], sga[k])

    def stage_nidx(c, k):
        buf = adjc[k]
        for r in range(CH):
            nidx_all[2 * c + r // 8, pl.ds((r % 8) * DEG, DEG)] = \
                buf[r, pl.ds(0, DEG)]

    def gA_dma(c, k):
        return pltpu.make_async_copy(emb_h.at[nidx_all.at[2 * c]], nbA[k],
                                     sgA[k])

    def gB_dma(c, k):
        return pltpu.make_async_copy(emb_h.at[nidx_all.at[2 * c + 1]], nbB[k],
                                     sgB[k])

    def gs_dma(c, k):
        return pltpu.make_async_copy(emb_h.at[ids(c)], sbuf[k], sgs[k])

    def wm_dma(c, k):
        return pltpu.make_async_copy(
            mbuf[k], nsum_h.at[pl.ds(base + c * CH, CH)], swm[k])

    def ws_dma(c, k):
        return pltpu.make_async_copy(
            sbuf[k], self_h.at[pl.ds(base + c * CH, CH)], sws[k])

    def reduce_chunk(k):
        a_buf, b_buf, mb = nbA[k], nbB[k], mbuf[k]
        bc = lambda v: lax.bitcast_convert_type(v, jnp.float32)

        def row(r, carry):
            for nb, ro in ((a_buf, 0), (b_buf, 8)):
                ws = [nb[r * DEG, pl.ds(kk * 16, 16)] for kk in range(8)]
                alo = [bc(w << 16) for w in ws]
                ahi = [bc(w) for w in ws]
                for j in range(1, DEG):
                    ws = [nb[r * DEG + j, pl.ds(kk * 16, 16)]
                          for kk in range(8)]
                    alo = [a + bc(w << 16) for a, w in zip(alo, ws)]
                    ahi = [a + bc(w) for a, w in zip(ahi, ws)]
                for kk in range(8):
                    mb[r + ro, pl.ds(kk * 16, 16)] = alo[kk]
                    mb[r + ro, pl.ds(128 + kk * 16, 16)] = ahi[kk]
            return carry
        lax.fori_loop(0, 8, row, 0)

    adj_dma(0, 0).start()
    adj_dma(0, 0).wait()
    stage_nidx(0, 0)
    adj_dma(1, 1).start()
    gA_dma(0, 0).start()
    gB_dma(0, 0).start()
    gs_dma(0, 0).start()

    def body(c, k):
        kn = 1 - k
        @pl.when(c + 1 < nch)
        def _():
            adj_dma(c + 1, kn).wait()
            stage_nidx(c + 1, kn)

            @pl.when(c >= 1)
            def _():
                ws_dma(c - 1, kn).wait()
            gA_dma(c + 1, kn).start()
            gB_dma(c + 1, kn).start()
            gs_dma(c + 1, kn).start()

        @pl.when(c + 2 < nch)
        def _():
            adj_dma(c + 2, k).start()

        gA_dma(c, k).wait()
        gB_dma(c, k).wait()
        gs_dma(c, k).wait()
        ws_dma(c, k).start()

        @pl.when(c >= 2)
        def _():
            wm_dma(c - 2, k).wait()
        reduce_chunk(k)
        wm_dma(c, k).start()

    def phase2(i, carry):
        c = i * 2
        body(c, 0)
        body(c + 1, 1)
        return carry

    lax.fori_loop(0, nch // 2, phase2, 0)

    wm_dma(nch - 2, 0).wait()
    ws_dma(nch - 2, 0).wait()
    wm_dma(nch - 1, 1).wait()
    ws_dma(nch - 1, 1).wait()


def _sc_gather(nodes_p, adj_p, emb_i):
    mesh = plsc.VectorSubcoreMesh(core_axis_name="c", subcore_axis_name="s")
    kern = functools.partial(
        pl.kernel,
        mesh=mesh,
        out_type=(
            jax.ShapeDtypeStruct((BP, 128), jnp.int32),
            jax.ShapeDtypeStruct((BP, D), jnp.float32),
        ),
        scratch_types=[
            pltpu.VMEM((NCH1 * CH,), jnp.int32),
            pltpu.VMEM((CH, 128), jnp.int32),
            pltpu.VMEM((CH, 128), jnp.int32),
            pltpu.VMEM((2 * NCH1, 128), jnp.int32),
            pltpu.VMEM((CH, 128), jnp.int32),
            pltpu.VMEM((CH, 128), jnp.int32),
            pltpu.VMEM((128, 128), jnp.int32),
            pltpu.VMEM((128, 128), jnp.int32),
            pltpu.VMEM((128, 128), jnp.int32),
            pltpu.VMEM((128, 128), jnp.int32),
            pltpu.VMEM((CH, D), jnp.float32),
            pltpu.VMEM((CH, D), jnp.float32),
        ] + [pltpu.SemaphoreType.DMA] * 12,
    )(_sc_gather_kernel)
    return kern(nodes_p, adj_p, emb_i)


def _mm_kernel(x1_ref, x2_ref, w_ref, b_ref, o_ref):
    x = x1_ref[...]
    xlo = lax.bitcast_convert_type(x << 16, jnp.float32).astype(jnp.bfloat16)
    xhi = lax.bitcast_convert_type(x & jnp.int32(-65536), jnp.float32
                                   ).astype(jnp.bfloat16)
    w = w_ref[...]
    acc = jnp.dot(xlo, w[:D // 2].astype(jnp.bfloat16),
                  preferred_element_type=jnp.float32)
    acc += jnp.dot(xhi, w[D // 2:D].astype(jnp.bfloat16),
                   preferred_element_type=jnp.float32)
    acc += jnp.dot(x2_ref[...].astype(jnp.bfloat16),
                   (w[D:] * (1.0 / DEG)).astype(jnp.bfloat16),
                   preferred_element_type=jnp.float32)
    o_ref[...] = jnp.maximum(acc + b_ref[...], 0.0)


def _tc_matmul(self_i, nsum, w, b2):
    bm = 2000
    grid = (B // bm,)
    return pl.pallas_call(
        _mm_kernel,
        grid=grid,
        in_specs=[
            pl.BlockSpec((bm, D // 2), lambda i: (i, 0)),
            pl.BlockSpec((bm, D), lambda i: (i, 0)),
            pl.BlockSpec((2 * D, D), lambda i: (0, 0)),
            pl.BlockSpec((1, D), lambda i: (0, 0)),
        ],
        out_specs=pl.BlockSpec((bm, D), lambda i: (i, 0)),
        out_shape=jax.ShapeDtypeStruct((B, D), jnp.float32),
    )(self_i, nsum, w, b2)


def kernel(nodes, adj, emb, W, b):
    nodes_p = jnp.pad(nodes.astype(jnp.int32), (0, BP - B))
    adj_p = jnp.pad(adj.astype(jnp.int32), ((0, 0), (0, 128 - DEG)))
    u = lax.bitcast_convert_type(emb.astype(jnp.bfloat16), jnp.uint16)
    emb_i = (u[:, :D // 2].astype(jnp.uint32)
             | (u[:, D // 2:].astype(jnp.uint32) << 16)).astype(jnp.int32)
    self_i, nsum = _sc_gather(nodes_p, adj_p, emb_i)
    return _tc_matmul(self_i, nsum, W, b.reshape(1, D))


# speedup vs baseline: 1.0511x; 1.0511x over previous
import functools

import jax
import jax.numpy as jnp
from jax import lax
from jax.experimental import pallas as pl
from jax.experimental.pallas import tpu as pltpu
from jax.experimental.pallas import tpu_sc as plsc

N_NODES = 10000
DEG = 16
D = 256
B = 10000
BP = 10240
NC = 2
NS = 16
NW = NC * NS
BPW = BP // NW
CH = 16
NCHUNK = BPW // CH
NCH0 = 22
NCH1 = 2 * NCHUNK - NCH0
NCHMAX = max(NCH0, NCH1)
BPS = BP + (NCHMAX * CH - BPW)


def _sc_gather_kernel(nodes_h, adj_h, emb_h, self_h, nsum_h,
                      idxc, adjc0, adjc1, nidx_all,
                      sbuf0, sbuf1, nbA0, nbA1, nbB0, nbB1, mbuf0, mbuf1,
                      sga0, sga1, sgA0, sgA1, sgB0, sgB1, sgs0, sgs1,
                      swm0, swm1, sws0, sws1):
    cid = lax.axis_index("c")
    sid = lax.axis_index("s")
    nch = jnp.where(cid == 0, NCH0, NCH1)
    base = sid * (2 * BPW) + cid * (NCH0 * CH)

    adjc = (adjc0, adjc1)
    sbuf = (sbuf0, sbuf1)
    nbA = (nbA0, nbA1)
    nbB = (nbB0, nbB1)
    mbuf = (mbuf0, mbuf1)
    sga = (sga0, sga1)
    sgA = (sgA0, sgA1)
    sgB = (sgB0, sgB1)
    sgs = (sgs0, sgs1)
    swm = (swm0, swm1)
    sws = (sws0, sws1)

    pltpu.sync_copy(nodes_h.at[pl.ds(base, NCHMAX * CH)], idxc)

    def ids(c):
        return idxc.at[pl.ds(c * CH, CH)]

    def adj_dma(c, k):
        return pltpu.make_async_copy(adj_h.at[ids(c)], adjc[k], sga[k])

    def stage_nidx(c, k):
        buf = adjc[k]
        for r in range(CH):
            nidx_all[2 * c + r // 8, pl.ds((r % 8) * DEG, DEG)] = \
                buf[r, pl.ds(0, DEG)]

    def gA_dma(c, k):
        return pltpu.make_async_copy(emb_h.at[nidx_all.at[2 * c]], nbA[k],
                                     sgA[k])

    def gB_dma(c, k):
        return pltpu.make_async_copy(emb_h.at[nidx_all.at[2 * c + 1]], nbB[k],
                                     sgB[k])

    def gs_dma(c, k):
        return pltpu.make_async_copy(emb_h.at[ids(c)], sbuf[k], sgs[k])

    def wm_dma(c, k):
        return pltpu.make_async_copy(
            mbuf[k], nsum_h.at[pl.ds(base + c * CH, CH)], swm[k])

    def ws_dma(c, k):
        return pltpu.make_async_copy(
            sbuf[k], self_h.at[pl.ds(base + c * CH, CH)], sws[k])

    def reduce_chunk(k):
        a_buf, b_buf, mb = nbA[k], nbB[k], mbuf[k]
        bc = lambda v: lax.bitcast_convert_type(v, jnp.float32)

        def row(r, carry):
            for nb, ro in ((a_buf, 0), (b_buf, 8)):
                ws = [nb[r * DEG, pl.ds(kk * 16, 16)] for kk in range(8)]
                alo = [bc(w << 16) for w in ws]
                ahi = [bc(w) for w in ws]
                for j in range(1, DEG):
                    ws = [nb[r * DEG + j, pl.ds(kk * 16, 16)]
                          for kk in range(8)]
                    alo = [a + bc(w << 16) for a, w in zip(alo, ws)]
                    ahi = [a + bc(w) for a, w in zip(ahi, ws)]
                for kk in range(8):
                    mb[r + ro, pl.ds(kk * 16, 16)] = alo[kk]
                    mb[r + ro, pl.ds(128 + kk * 16, 16)] = ahi[kk]
            return carry
        lax.fori_loop(0, 8, row, 0)

    adj_dma(0, 0).start()
    adj_dma(0, 0).wait()
    stage_nidx(0, 0)
    adj_dma(1, 1).start()
    gA_dma(0, 0).start()
    gB_dma(0, 0).start()
    gs_dma(0, 0).start()

    def body(c, k):
        kn = 1 - k
        @pl.when(c + 1 < nch)
        def _():
            adj_dma(c + 1, kn).wait()
            stage_nidx(c + 1, kn)

            @pl.when(c >= 1)
            def _():
                ws_dma(c - 1, kn).wait()
            gA_dma(c + 1, kn).start()
            gB_dma(c + 1, kn).start()
            gs_dma(c + 1, kn).start()

        @pl.when(c + 2 < nch)
        def _():
            adj_dma(c + 2, k).start()

        gA_dma(c, k).wait()
        gB_dma(c, k).wait()
        gs_dma(c, k).wait()
        ws_dma(c, k).start()

        @pl.when(c >= 2)
        def _():
            wm_dma(c - 2, k).wait()
        reduce_chunk(k)
        wm_dma(c, k).start()

    def phase2(i, carry):
        c = i * 2
        body(c, 0)
        body(c + 1, 1)
        return carry

    lax.fori_loop(0, nch // 2, phase2, 0)

    wm_dma(nch - 2, 0).wait()
    ws_dma(nch - 2, 0).wait()
    wm_dma(nch - 1, 1).wait()
    ws_dma(nch - 1, 1).wait()


def _sc_gather(nodes_p, adj_p, emb_i):
    mesh = plsc.VectorSubcoreMesh(core_axis_name="c", subcore_axis_name="s")
    kern = functools.partial(
        pl.kernel,
        mesh=mesh,
        out_type=(
            jax.ShapeDtypeStruct((BP, 128), jnp.int32),
            jax.ShapeDtypeStruct((BP, D), jnp.float32),
        ),
        scratch_types=[
            pltpu.VMEM((NCHMAX * CH,), jnp.int32),
            pltpu.VMEM((CH, 128), jnp.int32),
            pltpu.VMEM((CH, 128), jnp.int32),
            pltpu.VMEM((2 * NCHMAX, 128), jnp.int32),
            pltpu.VMEM((CH, 128), jnp.int32),
            pltpu.VMEM((CH, 128), jnp.int32),
            pltpu.VMEM((128, 128), jnp.int32),
            pltpu.VMEM((128, 128), jnp.int32),
            pltpu.VMEM((128, 128), jnp.int32),
            pltpu.VMEM((128, 128), jnp.int32),
            pltpu.VMEM((CH, D), jnp.float32),
            pltpu.VMEM((CH, D), jnp.float32),
        ] + [pltpu.SemaphoreType.DMA] * 12,
    )(_sc_gather_kernel)
    return kern(nodes_p, adj_p, emb_i)


def _mm_kernel(x1_ref, x2_ref, w_ref, b_ref, o_ref):
    x = x1_ref[...]
    xlo = lax.bitcast_convert_type(x << 16, jnp.float32).astype(jnp.bfloat16)
    xhi = lax.bitcast_convert_type(x & jnp.int32(-65536), jnp.float32
                                   ).astype(jnp.bfloat16)
    w = w_ref[...]
    acc = jnp.dot(xlo, w[:D // 2].astype(jnp.bfloat16),
                  preferred_element_type=jnp.float32)
    acc += jnp.dot(xhi, w[D // 2:D].astype(jnp.bfloat16),
                   preferred_element_type=jnp.float32)
    acc += jnp.dot(x2_ref[...].astype(jnp.bfloat16),
                   (w[D:] * (1.0 / DEG)).astype(jnp.bfloat16),
                   preferred_element_type=jnp.float32)
    o_ref[...] = jnp.maximum(acc + b_ref[...], 0.0)


def _tc_matmul(self_i, nsum, w, b2):
    bm = 2000
    grid = (B // bm,)
    return pl.pallas_call(
        _mm_kernel,
        grid=grid,
        in_specs=[
            pl.BlockSpec((bm, D // 2), lambda i: (i, 0)),
            pl.BlockSpec((bm, D), lambda i: (i, 0)),
            pl.BlockSpec((2 * D, D), lambda i: (0, 0)),
            pl.BlockSpec((1, D), lambda i: (0, 0)),
        ],
        out_specs=pl.BlockSpec((bm, D), lambda i: (i, 0)),
        out_shape=jax.ShapeDtypeStruct((B, D), jnp.float32),
    )(self_i, nsum, w, b2)


def kernel(nodes, adj, emb, W, b):
    nodes_p = jnp.pad(nodes.astype(jnp.int32), (0, BPS - B))
    adj_p = jnp.pad(adj.astype(jnp.int32), ((0, 0), (0, 128 - DEG)))
    u = lax.bitcast_convert_type(emb.astype(jnp.bfloat16), jnp.uint16)
    emb_i = (u[:, :D // 2].astype(jnp.uint32)
             | (u[:, D // 2:].astype(jnp.uint32) << 16)).astype(jnp.int32)
    self_i, nsum = _sc_gather(nodes_p, adj_p, emb_i)
    return _tc_matmul(self_i, nsum, W, b.reshape(1, D))
